# Initial kernel scaffold; baseline (speedup 1.0000x reference)
import jax
import jax.numpy as jnp
from jax.experimental import pallas as pl


def kernel(x, d, weight1, bias1, weight2, bias2, weight3, bias3, weight4, bias4, weight5, bias5):
    raise NotImplementedError("write your pallas kernel here")


# speedup vs baseline: 1.1553x; 1.1553x over previous
import functools

import jax
import jax.numpy as jnp
from jax import lax
from jax.experimental import pallas as pl
from jax.experimental.pallas import tpu as pltpu

N = 16
L_LOC = 10
L_DIR = 4
SCALE = 3.0
B = 32768
T = 128
NTILES = B // T
NCELL = N * N * N
W_ITEMS = NTILES + NCELL + 1

DX = 63
DD = 27
TE = 512


def _enc_body(x_ref, d_ref, ex_ref, ed_ref):
    def encode(vec, ncols):
        c = lax.broadcasted_iota(jnp.int32, (TE, ncols), 1)
        t = c - 3
        j = t // 6
        r = t - j * 6
        axis = jnp.where(c < 3, c, r % 3)
        x0 = vec[:, 0:1]
        x1 = vec[:, 1:2]
        x2 = vec[:, 2:3]
        xsel = jnp.where(axis == 0, x0, jnp.where(axis == 1, x1, x2))
        scale = lax.bitcast_convert_type((j + 127) << 23, jnp.float32)
        arg = xsel * scale
        val = jnp.where(r < 3, jnp.sin(arg), jnp.cos(arg))
        return jnp.where(c < 3, xsel, val)

    ex_ref[...] = encode(x_ref[...], DX)
    ed_ref[...] = encode(d_ref[...], DD)


def _mlp_body(s_tile, s_cid, s_s, s_e, s_first,
              ex_ref, ed_ref,
              w1_ref, b1_ref, w2_ref, b2_ref, w3_ref, b3_ref,
              w4_ref, b4_ref, w5_ref, b5_ref, out_ref):
    w = pl.program_id(0)
    seg_s = s_s[w]
    seg_e = s_e[w]
    first = s_first[w]
    cid = s_cid[w]

    riota = lax.broadcasted_iota(jnp.int32, (T, 8), 0)
    valid = (riota >= seg_s) & (riota < seg_e) & (cid < NCELL)

    f32 = jnp.float32
    a1 = ex_ref[...]
    w1 = w1_ref[0, 0, 0]
    h1 = jnp.maximum(jnp.dot(a1, w1, preferred_element_type=f32)
                     + b1_ref[0, 0, 0], 0.0)
    o2 = jnp.maximum(jnp.dot(h1, w2_ref[0, 0, 0], preferred_element_type=f32)
                     + b2_ref[0, 0, 0], 0.0)
    sigma = o2[:, 0:1]
    h2 = o2[:, 1:33]
    h3 = (jnp.dot(h2, w3_ref[0, 0, 0], preferred_element_type=f32)
          + b3_ref[0, 0, 0])
    a4 = jnp.concatenate([h3, ed_ref[...]], axis=1)
    h4 = jnp.maximum(jnp.dot(a4, w4_ref[0, 0, 0], preferred_element_type=f32)
                     + b4_ref[0, 0, 0], 0.0)
    o5 = jax.nn.sigmoid(jnp.dot(h4, w5_ref[0, 0, 0], preferred_element_type=f32)
                        + b5_ref[0, 0, 0])
    vals = jnp.concatenate([o5, sigma, jnp.zeros((T, 4), f32)], axis=1)

    base = jnp.where(first > 0, jnp.zeros((T, 8), f32), out_ref[...])
    out_ref[...] = jnp.where(valid, vals, base)


def _cell_index(s_cid, w):
    cid = jnp.minimum(s_cid[w], NCELL - 1)
    return cid // (N * N), (cid // N) % N, cid % N


def kernel(x, d, weight1, bias1, weight2, bias2, weight3, bias3,
           weight4, bias4, weight5, bias5):
    half = SCALE / 2
    mask = (jnp.abs(x[:, 0]) < half) & (jnp.abs(x[:, 1]) < half) & (jnp.abs(x[:, 2]) < half)
    i = jnp.clip((x / (SCALE / N) + N / 2).astype(jnp.int32), 0, N - 1)
    c = (i[:, 0] * N + i[:, 1]) * N + i[:, 2]
    c = jnp.where(mask, c, NCELL).astype(jnp.int32)

    order = jnp.argsort(c).astype(jnp.int32)
    cs = c[order]
    xs = x[order]
    ds = d[order]

    row = jnp.arange(B, dtype=jnp.int32)
    prev = jnp.concatenate([jnp.full((1,), -1, jnp.int32), cs[:-1]])
    is_new = ((row % T) == 0) | (cs != prev)
    item = jnp.cumsum(is_new.astype(jnp.int32)) - 1

    bidx = jnp.where(is_new, item, W_ITEMS)
    s_tile = jnp.full((W_ITEMS + 1,), NTILES - 1, jnp.int32).at[bidx].set(row // T)[:W_ITEMS]
    s_cid = jnp.full((W_ITEMS + 1,), NCELL, jnp.int32).at[bidx].set(cs)[:W_ITEMS]
    s_s = jnp.zeros((W_ITEMS + 1,), jnp.int32).at[bidx].set(row % T)[:W_ITEMS]
    s_e = jnp.zeros((W_ITEMS + 1,), jnp.int32).at[item].max((row % T) + 1)[:W_ITEMS]
    s_first = ((s_s == 0) & (s_e > 0)).astype(jnp.int32)

    enc = pl.pallas_call(
        _enc_body,
        grid=(B // TE,),
        in_specs=[pl.BlockSpec((TE, 3), lambda t: (t, 0)),
                  pl.BlockSpec((TE, 3), lambda t: (t, 0))],
        out_specs=[pl.BlockSpec((TE, DX), lambda t: (t, 0)),
                   pl.BlockSpec((TE, DD), lambda t: (t, 0))],
        out_shape=[jax.ShapeDtypeStruct((B, DX), jnp.float32),
                   jax.ShapeDtypeStruct((B, DD), jnp.float32)],
    )
    ex, ed = enc(xs, ds)

    def wspec(k, n):
        return pl.BlockSpec(
            (1, 1, 1, k, n),
            lambda w, st, sc, ss, se, sf: (*_cell_index(sc, w), 0, 0))

    grid_spec = pltpu.PrefetchScalarGridSpec(
        num_scalar_prefetch=5,
        grid=(W_ITEMS,),
        in_specs=[
            pl.BlockSpec((T, DX), lambda w, st, sc, ss, se, sf: (st[w], 0)),
            pl.BlockSpec((T, DD), lambda w, st, sc, ss, se, sf: (st[w], 0)),
            wspec(63, 32), wspec(1, 32),
            wspec(32, 33), wspec(1, 33),
            wspec(32, 32), wspec(1, 32),
            wspec(59, 32), wspec(1, 32),
            wspec(32, 3), wspec(1, 3),
        ],
        out_specs=pl.BlockSpec((T, 8), lambda w, st, sc, ss, se, sf: (st[w], 0)),
    )
    out_s = pl.pallas_call(
        _mlp_body,
        grid_spec=grid_spec,
        out_shape=jax.ShapeDtypeStruct((B, 8), jnp.float32),
    )(s_tile, s_cid, s_s, s_e, s_first,
      ex, ed,
      weight1, bias1, weight2, bias2, weight3, bias3,
      weight4, bias4, weight5, bias5)

    out = jnp.zeros((B, 8), jnp.float32).at[order].set(out_s)
    return out[:, 0:3], out[:, 3:4]


# speedup vs baseline: 1.3598x; 1.1770x over previous
import functools

import jax
import jax.numpy as jnp
from jax import lax
from jax.experimental import pallas as pl
from jax.experimental.pallas import tpu as pltpu

N = 16
L_LOC = 10
L_DIR = 4
SCALE = 3.0
B = 32768
T = 256
NTILES = B // T
NCELL = N * N * N
W_ITEMS = NTILES + NCELL + 1

DX = 63
DD = 27
TE = 512


def _enc_body(x_ref, d_ref, ex_ref, ed_ref):
    def encode(vec, ncols):
        c = lax.broadcasted_iota(jnp.int32, (TE, ncols), 1)
        t = c - 3
        j = t // 6
        r = t - j * 6
        axis = jnp.where(c < 3, c, r % 3)
        x0 = vec[:, 0:1]
        x1 = vec[:, 1:2]
        x2 = vec[:, 2:3]
        xsel = jnp.where(axis == 0, x0, jnp.where(axis == 1, x1, x2))
        scale = lax.bitcast_convert_type((j + 127) << 23, jnp.float32)
        arg = xsel * scale
        val = jnp.where(r < 3, jnp.sin(arg), jnp.cos(arg))
        return jnp.where(c < 3, xsel, val)

    ex_ref[...] = encode(x_ref[...], DX)
    ed_ref[...] = encode(d_ref[...], DD)


def _mlp_body(s_tile, s_cid, s_s, s_e, s_first,
              ex_ref, ed_ref, bb32_ref, bb33_ref, out_ref):
    w = pl.program_id(0)
    seg_s = s_s[w]
    seg_e = s_e[w]
    first = s_first[w]
    cid = s_cid[w]

    riota = lax.broadcasted_iota(jnp.int32, (T, 8), 0)
    valid = (riota >= seg_s) & (riota < seg_e) & (cid < NCELL)

    f32 = jnp.float32
    bb = bb32_ref[0]
    w1 = bb[0:64]
    w3 = bb[64:104]
    w4 = bb[104:168]
    w5 = bb[168:208]
    w2 = bb33_ref[0]

    ones = jnp.ones((T, 1), f32)
    z7 = jnp.zeros((T, 7), f32)
    dot = functools.partial(jnp.dot, preferred_element_type=f32)

    a1 = jnp.concatenate([ex_ref[...], ones], axis=1)
    h1 = jnp.maximum(dot(a1, w1), 0.0)
    a2 = jnp.concatenate([h1, ones, z7], axis=1)
    o2 = jnp.maximum(dot(a2, w2), 0.0)
    h2 = o2[:, 0:32]
    sigma = o2[:, 32:33]
    a3 = jnp.concatenate([h2, ones, z7], axis=1)
    h3 = dot(a3, w3)
    a4 = jnp.concatenate([h3, ed_ref[...], ones,
                          jnp.zeros((T, 4), f32)], axis=1)
    h4 = jnp.maximum(dot(a4, w4), 0.0)
    o5 = jax.nn.sigmoid(dot(a5 := jnp.concatenate([h4, ones, z7], axis=1), w5))
    vals = jnp.concatenate([o5[:, 0:3], sigma,
                            jnp.zeros((T, 4), f32)], axis=1)

    base = jnp.where(first > 0, jnp.zeros((T, 8), f32), out_ref[...])
    out_ref[...] = jnp.where(valid, vals, base)


def kernel(x, d, weight1, bias1, weight2, bias2, weight3, bias3,
           weight4, bias4, weight5, bias5):
    f32 = jnp.float32
    w1r = weight1.reshape(NCELL, 63, 32)
    b1r = bias1.reshape(NCELL, 1, 32)
    w3r = weight3.reshape(NCELL, 32, 32)
    b3r = bias3.reshape(NCELL, 1, 32)
    w4r = weight4.reshape(NCELL, 59, 32)
    b4r = bias4.reshape(NCELL, 1, 32)
    w5p = jnp.pad(weight5.reshape(NCELL, 32, 3), ((0, 0), (0, 0), (0, 29)))
    b5p = jnp.pad(bias5.reshape(NCELL, 1, 3), ((0, 0), (0, 0), (0, 29)))
    z = lambda n: jnp.zeros((NCELL, n, 32), f32)
    bb32 = jnp.concatenate(
        [w1r, b1r, w3r, b3r, z(7), w4r, b4r, z(4), w5p, b5p, z(7)], axis=1)
    w2r = weight2.reshape(NCELL, 32, 33)
    b2r = bias2.reshape(NCELL, 1, 33)
    w2s = jnp.concatenate([w2r[:, :, 1:33], w2r[:, :, 0:1]], axis=2)
    b2s = jnp.concatenate([b2r[:, :, 1:33], b2r[:, :, 0:1]], axis=2)
    bb33 = jnp.concatenate([w2s, b2s, jnp.zeros((NCELL, 7, 33), f32)], axis=1)

    half = SCALE / 2
    mask = (jnp.abs(x[:, 0]) < half) & (jnp.abs(x[:, 1]) < half) & (jnp.abs(x[:, 2]) < half)
    i = jnp.clip((x / (SCALE / N) + N / 2).astype(jnp.int32), 0, N - 1)
    c = (i[:, 0] * N + i[:, 1]) * N + i[:, 2]
    c = jnp.where(mask, c, NCELL).astype(jnp.int32)

    order = jnp.argsort(c).astype(jnp.int32)
    cs = c[order]
    xs = x[order]
    ds = d[order]

    row = jnp.arange(B, dtype=jnp.int32)
    prev = jnp.concatenate([jnp.full((1,), -1, jnp.int32), cs[:-1]])
    is_new = ((row % T) == 0) | (cs != prev)
    item = jnp.cumsum(is_new.astype(jnp.int32)) - 1

    bidx = jnp.where(is_new, item, W_ITEMS)
    s_tile = jnp.full((W_ITEMS + 1,), NTILES - 1, jnp.int32).at[bidx].set(row // T)[:W_ITEMS]
    s_cid = jnp.full((W_ITEMS + 1,), NCELL, jnp.int32).at[bidx].set(cs)[:W_ITEMS]
    s_s = jnp.zeros((W_ITEMS + 1,), jnp.int32).at[bidx].set(row % T)[:W_ITEMS]
    s_e = jnp.zeros((W_ITEMS + 1,), jnp.int32).at[item].max((row % T) + 1)[:W_ITEMS]
    s_first = ((s_s == 0) & (s_e > 0)).astype(jnp.int32)

    enc = pl.pallas_call(
        _enc_body,
        grid=(B // TE,),
        in_specs=[pl.BlockSpec((TE, 3), lambda t: (t, 0)),
                  pl.BlockSpec((TE, 3), lambda t: (t, 0))],
        out_specs=[pl.BlockSpec((TE, DX), lambda t: (t, 0)),
                   pl.BlockSpec((TE, DD), lambda t: (t, 0))],
        out_shape=[jax.ShapeDtypeStruct((B, DX), jnp.float32),
                   jax.ShapeDtypeStruct((B, DD), jnp.float32)],
    )
    ex, ed = enc(xs, ds)

    grid_spec = pltpu.PrefetchScalarGridSpec(
        num_scalar_prefetch=5,
        grid=(W_ITEMS,),
        in_specs=[
            pl.BlockSpec((T, DX), lambda w, st, sc, ss, se, sf: (st[w], 0)),
            pl.BlockSpec((T, DD), lambda w, st, sc, ss, se, sf: (st[w], 0)),
            pl.BlockSpec((1, 208, 32),
                         lambda w, st, sc, ss, se, sf:
                         (jnp.minimum(sc[w], NCELL - 1), 0, 0)),
            pl.BlockSpec((1, 40, 33),
                         lambda w, st, sc, ss, se, sf:
                         (jnp.minimum(sc[w], NCELL - 1), 0, 0)),
        ],
        out_specs=pl.BlockSpec((T, 8), lambda w, st, sc, ss, se, sf: (st[w], 0)),
    )
    out_s = pl.pallas_call(
        _mlp_body,
        grid_spec=grid_spec,
        out_shape=jax.ShapeDtypeStruct((B, 8), jnp.float32),
    )(s_tile, s_cid, s_s, s_e, s_first, ex, ed, bb32, bb33)

    out = jnp.zeros((B, 8), jnp.float32).at[order].set(out_s)
    return out[:, 0:3], out[:, 3:4]


# speedup vs baseline: 1.5083x; 1.1092x over previous
import functools

import jax
import jax.numpy as jnp
from jax import lax
from jax.experimental import pallas as pl
from jax.experimental.pallas import tpu as pltpu

N = 16
SCALE = 3.0
B = 32768
T = 256
NTILES = B // T
NCELL = N * N * N
W_ITEMS = NTILES + NCELL + 1

DX = 63
DD = 27
TE = 512


def _enc_body(x_ref, d_ref, ex_ref, ed_ref):
    def encode(vec, ncols):
        c = lax.broadcasted_iota(jnp.int32, (TE, ncols), 1)
        t = c - 3
        j = t // 6
        r = t - j * 6
        axis = jnp.where(c < 3, c, r % 3)
        x0 = vec[:, 0:1]
        x1 = vec[:, 1:2]
        x2 = vec[:, 2:3]
        xsel = jnp.where(axis == 0, x0, jnp.where(axis == 1, x1, x2))
        scale = lax.bitcast_convert_type((j + 127) << 23, jnp.float32)
        arg = xsel * scale
        val = jnp.where(r < 3, jnp.sin(arg), jnp.cos(arg))
        return jnp.where(c < 3, xsel, val)

    ex_ref[...] = encode(x_ref[...], DX)
    ed_ref[...] = encode(d_ref[...], DD)


def _mlp_body(s_tile, s_cid, s_s, s_e, s_first,
              ex_ref, ed_ref, bb_ref, out_ref):
    w = pl.program_id(0)
    seg_s = s_s[w]
    seg_e = s_e[w]
    first = s_first[w]
    cid = s_cid[w]

    riota = lax.broadcasted_iota(jnp.int32, (T, 8), 0)
    valid = (riota >= seg_s) & (riota < seg_e) & (cid < NCELL)

    f32 = jnp.float32
    big = bb_ref[0]
    w1 = big[0:64, 0:32]
    w4 = big[0:64, 32:64]
    w2f = big[0:40, 64:96]
    w3 = big[40:80, 64:96]
    sk = big[0:40, 96:104]

    ones = jnp.ones((T, 1), f32)
    z7 = jnp.zeros((T, 7), f32)
    dot = functools.partial(jnp.dot, preferred_element_type=f32)

    a1 = jnp.concatenate([ex_ref[...], ones], axis=1)
    h1 = jnp.maximum(dot(a1, w1), 0.0)
    a2 = jnp.concatenate([h1, ones, z7], axis=1)
    h2 = jnp.maximum(dot(a2, w2f), 0.0)
    sig = jnp.maximum(dot(a2, sk)[:, 3:4], 0.0)
    a3 = jnp.concatenate([h2, ones, z7], axis=1)
    h3 = dot(a3, w3)
    a4 = jnp.concatenate([h3, ed_ref[...], ones,
                          jnp.zeros((T, 4), f32)], axis=1)
    h4 = jnp.maximum(dot(a4, w4), 0.0)
    a5 = jnp.concatenate([h4, ones, z7], axis=1)
    o5 = jax.nn.sigmoid(dot(a5, sk))
    vals = jnp.concatenate([o5[:, 0:3], sig,
                            jnp.zeros((T, 4), f32)], axis=1)

    base = jnp.where(first > 0, jnp.zeros((T, 8), f32), out_ref[...])
    out_ref[...] = jnp.where(valid, vals, base)


def kernel(x, d, weight1, bias1, weight2, bias2, weight3, bias3,
           weight4, bias4, weight5, bias5):
    f32 = jnp.float32
    w1r = weight1.reshape(NCELL, 63, 32)
    b1r = bias1.reshape(NCELL, 1, 32)
    w2r = weight2.reshape(NCELL, 32, 33)
    b2r = bias2.reshape(NCELL, 1, 33)
    w3r = weight3.reshape(NCELL, 32, 32)
    b3r = bias3.reshape(NCELL, 1, 32)
    w4r = weight4.reshape(NCELL, 59, 32)
    b4r = bias4.reshape(NCELL, 1, 32)
    w5r = weight5.reshape(NCELL, 32, 3)
    b5r = bias5.reshape(NCELL, 1, 3)

    def z(nrow, nlane):
        return jnp.zeros((NCELL, nrow, nlane), f32)

    p0 = jnp.concatenate([w1r, b1r, z(16, 32)], axis=1)
    p1 = jnp.concatenate([w4r, b4r, z(20, 32)], axis=1)
    p2 = jnp.concatenate([w2r[:, :, 1:33], b2r[:, :, 1:33], z(7, 32),
                          w3r, b3r, z(7, 32)], axis=1)
    skw = jnp.concatenate([w5r, w2r[:, :, 0:1], jnp.zeros((NCELL, 32, 4), f32)],
                          axis=2)
    skb = jnp.concatenate([b5r, b2r[:, :, 0:1], jnp.zeros((NCELL, 1, 4), f32)],
                          axis=2)
    p3 = jnp.concatenate([skw, skb, z(7, 8), z(40, 8)], axis=1)
    big = jnp.concatenate([p0, p1, p2, p3, z(80, 24)], axis=2)

    half = SCALE / 2
    mask = (jnp.abs(x[:, 0]) < half) & (jnp.abs(x[:, 1]) < half) & (jnp.abs(x[:, 2]) < half)
    i = jnp.clip((x / (SCALE / N) + N / 2).astype(jnp.int32), 0, N - 1)
    c = (i[:, 0] * N + i[:, 1]) * N + i[:, 2]
    c = jnp.where(mask, c, NCELL).astype(jnp.int32)

    row = jnp.arange(B, dtype=jnp.int32)
    cs, order = lax.sort((c, row), num_keys=1)
    xs = x[order]
    ds = d[order]

    prev = jnp.concatenate([jnp.full((1,), -1, jnp.int32), cs[:-1]])
    is_new = ((row % T) == 0) | (cs != prev)
    bpos = jnp.where(is_new, row, B)
    rstart = lax.sort(bpos)[:W_ITEMS]
    rnext = jnp.concatenate([rstart[1:], jnp.full((1,), B, jnp.int32)])
    valid_it = rstart < B
    s_tile = jnp.minimum(rstart // T, NTILES - 1)
    s_s = rstart % T
    s_e = jnp.where(valid_it, ((rnext - 1) % T) + 1, 0)
    s_cid = cs[jnp.minimum(rstart, B - 1)]
    s_first = ((s_s == 0) & valid_it).astype(jnp.int32)

    enc = pl.pallas_call(
        _enc_body,
        grid=(B // TE,),
        in_specs=[pl.BlockSpec((TE, 3), lambda t: (t, 0)),
                  pl.BlockSpec((TE, 3), lambda t: (t, 0))],
        out_specs=[pl.BlockSpec((TE, DX), lambda t: (t, 0)),
                   pl.BlockSpec((TE, DD), lambda t: (t, 0))],
        out_shape=[jax.ShapeDtypeStruct((B, DX), jnp.float32),
                   jax.ShapeDtypeStruct((B, DD), jnp.float32)],
    )
    ex, ed = enc(xs, ds)

    grid_spec = pltpu.PrefetchScalarGridSpec(
        num_scalar_prefetch=5,
        grid=(W_ITEMS,),
        in_specs=[
            pl.BlockSpec((T, DX), lambda w, st, sc, ss, se, sf: (st[w], 0)),
            pl.BlockSpec((T, DD), lambda w, st, sc, ss, se, sf: (st[w], 0)),
            pl.BlockSpec((1, 80, 128),
                         lambda w, st, sc, ss, se, sf:
                         (jnp.minimum(sc[w], NCELL - 1), 0, 0)),
        ],
        out_specs=pl.BlockSpec((T, 8), lambda w, st, sc, ss, se, sf: (st[w], 0)),
    )
    out_s = pl.pallas_call(
        _mlp_body,
        grid_spec=grid_spec,
        out_shape=jax.ShapeDtypeStruct((B, 8), jnp.float32),
    )(s_tile, s_cid, s_s, s_e, s_first, ex, ed, big)

    out = jnp.zeros((B, 8), jnp.float32).at[order].set(out_s)
    return out[:, 0:3], out[:, 3:4]


# speedup vs baseline: 1.8955x; 1.2567x over previous
import functools

import jax
import jax.numpy as jnp
from jax import lax
from jax.experimental import pallas as pl
from jax.experimental.pallas import tpu as pltpu

N = 16
SCALE = 3.0
B = 32768
T = 256
NTILES = B // T
NCELL = N * N * N
W_ITEMS = NTILES + NCELL + 1

DX = 63
DD = 27
TE = 512


def _enc_body(x_ref, d_ref, ex_ref, ed_ref):
    def encode(vec, ncols):
        c = lax.broadcasted_iota(jnp.int32, (TE, ncols), 1)
        t = c - 3
        j = t // 6
        r = t - j * 6
        axis = jnp.where(c < 3, c, r % 3)
        x0 = vec[:, 0:1]
        x1 = vec[:, 1:2]
        x2 = vec[:, 2:3]
        xsel = jnp.where(axis == 0, x0, jnp.where(axis == 1, x1, x2))
        scale = lax.bitcast_convert_type((j + 127) << 23, jnp.float32)
        arg = xsel * scale
        val = jnp.where(r < 3, jnp.sin(arg), jnp.cos(arg))
        return jnp.where(c < 3, xsel, val)

    ex_ref[...] = encode(x_ref[...], DX)
    ed_ref[...] = encode(d_ref[...], DD)


CH = 32


def _pack_body(w1, b1, w2, b2, w3, b3, w4, b4, w5, b5, out_ref):
    out_ref[...] = jnp.zeros((CH, 80, 128), jnp.float32)
    out_ref[:, 0:63, 0:32] = w1[...]
    out_ref[:, 63:64, 0:32] = b1[...]
    out_ref[:, 0:59, 32:64] = w4[...]
    out_ref[:, 59:60, 32:64] = b4[...]
    out_ref[:, 0:32, 64:96] = w2[:, :, 1:33]
    out_ref[:, 32:33, 64:96] = b2[:, :, 1:33]
    out_ref[:, 40:72, 64:96] = w3[...]
    out_ref[:, 72:73, 64:96] = b3[...]
    out_ref[:, 0:32, 96:99] = w5[...]
    out_ref[:, 32:33, 96:99] = b5[...]
    out_ref[:, 0:32, 99:100] = w2[:, :, 0:1]
    out_ref[:, 32:33, 99:100] = b2[:, :, 0:1]


def _mlp_body(s_tile, s_cid, s_s, s_e, s_first,
              ex_ref, ed_ref, bb_ref, out_ref):
    w = pl.program_id(0)
    seg_s = s_s[w]
    seg_e = s_e[w]
    first = s_first[w]
    cid = s_cid[w]

    riota = lax.broadcasted_iota(jnp.int32, (T, 8), 0)
    valid = (riota >= seg_s) & (riota < seg_e) & (cid < NCELL)

    f32 = jnp.float32
    big = bb_ref[0]
    w1 = big[0:64, 0:32]
    w4 = big[0:64, 32:64]
    w2f = big[0:40, 64:96]
    w3 = big[40:80, 64:96]
    sk = big[0:40, 96:104]

    ones = jnp.ones((T, 1), f32)
    z7 = jnp.zeros((T, 7), f32)
    dot = functools.partial(jnp.dot, preferred_element_type=f32)

    a1 = jnp.concatenate([ex_ref[...], ones], axis=1)
    h1 = jnp.maximum(dot(a1, w1), 0.0)
    a2 = jnp.concatenate([h1, ones, z7], axis=1)
    h2 = jnp.maximum(dot(a2, w2f), 0.0)
    sig = jnp.maximum(dot(a2, sk)[:, 3:4], 0.0)
    a3 = jnp.concatenate([h2, ones, z7], axis=1)
    h3 = dot(a3, w3)
    a4 = jnp.concatenate([h3, ed_ref[...], ones,
                          jnp.zeros((T, 4), f32)], axis=1)
    h4 = jnp.maximum(dot(a4, w4), 0.0)
    a5 = jnp.concatenate([h4, ones, z7], axis=1)
    o5 = jax.nn.sigmoid(dot(a5, sk))
    vals = jnp.concatenate([o5[:, 0:3], sig,
                            jnp.zeros((T, 4), f32)], axis=1)

    base = jnp.where(first > 0, jnp.zeros((T, 8), f32), out_ref[...])
    out_ref[...] = jnp.where(valid, vals, base)


def kernel(x, d, weight1, bias1, weight2, bias2, weight3, bias3,
           weight4, bias4, weight5, bias5):
    f32 = jnp.float32
    w1r = weight1.reshape(NCELL, 63, 32)
    b1r = bias1.reshape(NCELL, 1, 32)
    w2r = weight2.reshape(NCELL, 32, 33)
    b2r = bias2.reshape(NCELL, 1, 33)
    w3r = weight3.reshape(NCELL, 32, 32)
    b3r = bias3.reshape(NCELL, 1, 32)
    w4r = weight4.reshape(NCELL, 59, 32)
    b4r = bias4.reshape(NCELL, 1, 32)
    w5r = weight5.reshape(NCELL, 32, 3)
    b5r = bias5.reshape(NCELL, 1, 3)

    def pspec(k, n):
        return pl.BlockSpec((CH, k, n), lambda g: (g, 0, 0))

    big = pl.pallas_call(
        _pack_body,
        grid=(NCELL // CH,),
        in_specs=[pspec(63, 32), pspec(1, 32), pspec(32, 33), pspec(1, 33),
                  pspec(32, 32), pspec(1, 32), pspec(59, 32), pspec(1, 32),
                  pspec(32, 3), pspec(1, 3)],
        out_specs=pl.BlockSpec((CH, 80, 128), lambda g: (g, 0, 0)),
        out_shape=jax.ShapeDtypeStruct((NCELL, 80, 128), jnp.float32),
    )(w1r, b1r, w2r, b2r, w3r, b3r, w4r, b4r, w5r, b5r)

    half = SCALE / 2
    mask = (jnp.abs(x[:, 0]) < half) & (jnp.abs(x[:, 1]) < half) & (jnp.abs(x[:, 2]) < half)
    i = jnp.clip((x / (SCALE / N) + N / 2).astype(jnp.int32), 0, N - 1)
    c = (i[:, 0] * N + i[:, 1]) * N + i[:, 2]
    c = jnp.where(mask, c, NCELL).astype(jnp.int32)

    row = jnp.arange(B, dtype=jnp.int32)
    cs, order = lax.sort((c, row), num_keys=1)
    xs = x[order]
    ds = d[order]

    prev = jnp.concatenate([jnp.full((1,), -1, jnp.int32), cs[:-1]])
    is_new = ((row % T) == 0) | (cs != prev)
    bpos = jnp.where(is_new, row, B)
    rstart = lax.sort(bpos)[:W_ITEMS]
    rnext = jnp.concatenate([rstart[1:], jnp.full((1,), B, jnp.int32)])
    valid_it = rstart < B
    s_tile = jnp.minimum(rstart // T, NTILES - 1)
    s_s = rstart % T
    s_e = jnp.where(valid_it, ((rnext - 1) % T) + 1, 0)
    s_cid = cs[jnp.minimum(rstart, B - 1)]
    s_first = ((s_s == 0) & valid_it).astype(jnp.int32)

    enc = pl.pallas_call(
        _enc_body,
        grid=(B // TE,),
        in_specs=[pl.BlockSpec((TE, 3), lambda t: (t, 0)),
                  pl.BlockSpec((TE, 3), lambda t: (t, 0))],
        out_specs=[pl.BlockSpec((TE, DX), lambda t: (t, 0)),
                   pl.BlockSpec((TE, DD), lambda t: (t, 0))],
        out_shape=[jax.ShapeDtypeStruct((B, DX), jnp.float32),
                   jax.ShapeDtypeStruct((B, DD), jnp.float32)],
    )
    ex, ed = enc(xs, ds)

    grid_spec = pltpu.PrefetchScalarGridSpec(
        num_scalar_prefetch=5,
        grid=(W_ITEMS,),
        in_specs=[
            pl.BlockSpec((T, DX), lambda w, st, sc, ss, se, sf: (st[w], 0)),
            pl.BlockSpec((T, DD), lambda w, st, sc, ss, se, sf: (st[w], 0)),
            pl.BlockSpec((1, 80, 128),
                         lambda w, st, sc, ss, se, sf:
                         (jnp.minimum(sc[w], NCELL - 1), 0, 0)),
        ],
        out_specs=pl.BlockSpec((T, 8), lambda w, st, sc, ss, se, sf: (st[w], 0)),
    )
    out_s = pl.pallas_call(
        _mlp_body,
        grid_spec=grid_spec,
        out_shape=jax.ShapeDtypeStruct((B, 8), jnp.float32),
    )(s_tile, s_cid, s_s, s_e, s_first, ex, ed, big)

    out = jnp.zeros((B, 8), jnp.float32).at[order].set(out_s)
    return out[:, 0:3], out[:, 3:4]


# speedup vs baseline: 1.9462x; 1.0268x over previous
import functools

import jax
import jax.numpy as jnp
from jax import lax
from jax.experimental import pallas as pl
from jax.experimental.pallas import tpu as pltpu

N = 16
SCALE = 3.0
B = 32768
T = 256
NTILES = B // T
NCELL = N * N * N
W_ITEMS = NTILES + NCELL + 1

DX = 63
DD = 27
TE = 512


def _enc_body(x_ref, d_ref, ex_ref, ed_ref):
    def encode(vec, ncols):
        c = lax.broadcasted_iota(jnp.int32, (TE, ncols), 1)
        t = c - 3
        j = t // 6
        r = t - j * 6
        axis = jnp.where(c < 3, c, r % 3)
        x0 = vec[:, 0:1]
        x1 = vec[:, 1:2]
        x2 = vec[:, 2:3]
        xsel = jnp.where(axis == 0, x0, jnp.where(axis == 1, x1, x2))
        scale = lax.bitcast_convert_type((j + 127) << 23, jnp.float32)
        arg = xsel * scale
        val = jnp.where(r < 3, jnp.sin(arg), jnp.cos(arg))
        return jnp.where(c < 3, xsel, val)

    ex_ref[...] = encode(x_ref[...], DX)
    ed_ref[...] = encode(d_ref[...], DD)


CH = 32


def _pack_body(w1, b1, w2, b2, w3, b3, w4, b4, w5, b5, out_ref):
    out_ref[...] = jnp.zeros((CH, 80, 128), jnp.float32)
    out_ref[:, 0:63, 0:32] = w1[...]
    out_ref[:, 63:64, 0:32] = b1[...]
    out_ref[:, 0:59, 32:64] = w4[...]
    out_ref[:, 59:60, 32:64] = b4[...]
    out_ref[:, 0:32, 64:96] = w2[:, :, 1:33]
    out_ref[:, 32:33, 64:96] = b2[:, :, 1:33]
    out_ref[:, 40:72, 64:96] = w3[...]
    out_ref[:, 72:73, 64:96] = b3[...]
    out_ref[:, 0:32, 96:99] = w5[...]
    out_ref[:, 32:33, 96:99] = b5[...]
    out_ref[:, 0:32, 99:100] = w2[:, :, 0:1]
    out_ref[:, 32:33, 99:100] = b2[:, :, 0:1]


def _mlp_body(s_tile, s_cid, s_s, s_e, s_first,
              ex_ref, ed_ref, bb_ref, out_ref):
    w = pl.program_id(0)
    seg_s = s_s[w]
    seg_e = s_e[w]
    first = s_first[w]
    cid = s_cid[w]

    active = (s_first[w] > 0) | ((seg_e > seg_s) & (cid < NCELL))

    @pl.when(active)
    def _run():
        _mlp_tile(seg_s, seg_e, first, cid, ex_ref, ed_ref, bb_ref, out_ref)


def _mlp_tile(seg_s, seg_e, first, cid, ex_ref, ed_ref, bb_ref, out_ref):
    riota = lax.broadcasted_iota(jnp.int32, (T, 8), 0)
    valid = (riota >= seg_s) & (riota < seg_e) & (cid < NCELL)

    f32 = jnp.float32
    big = bb_ref[0]
    w1 = big[0:64, 0:32]
    w4 = big[0:64, 32:64]
    w2f = big[0:40, 64:96]
    w3 = big[40:80, 64:96]
    sk = big[0:40, 96:104]

    ones = jnp.ones((T, 1), f32)
    z7 = jnp.zeros((T, 7), f32)
    dot = functools.partial(jnp.dot, preferred_element_type=f32)

    a1 = jnp.concatenate([ex_ref[...], ones], axis=1)
    h1 = jnp.maximum(dot(a1, w1), 0.0)
    a2 = jnp.concatenate([h1, ones, z7], axis=1)
    h2 = jnp.maximum(dot(a2, w2f), 0.0)
    sig = jnp.maximum(dot(a2, sk)[:, 3:4], 0.0)
    a3 = jnp.concatenate([h2, ones, z7], axis=1)
    h3 = dot(a3, w3)
    a4 = jnp.concatenate([h3, ed_ref[...], ones,
                          jnp.zeros((T, 4), f32)], axis=1)
    h4 = jnp.maximum(dot(a4, w4), 0.0)
    a5 = jnp.concatenate([h4, ones, z7], axis=1)
    o5 = jax.nn.sigmoid(dot(a5, sk))
    vals = jnp.concatenate([o5[:, 0:3], sig,
                            jnp.zeros((T, 4), f32)], axis=1)

    base = jnp.where(first > 0, jnp.zeros((T, 8), f32), out_ref[...])
    out_ref[...] = jnp.where(valid, vals, base)


def kernel(x, d, weight1, bias1, weight2, bias2, weight3, bias3,
           weight4, bias4, weight5, bias5):
    f32 = jnp.float32
    w1r = weight1.reshape(NCELL, 63, 32)
    b1r = bias1.reshape(NCELL, 1, 32)
    w2r = weight2.reshape(NCELL, 32, 33)
    b2r = bias2.reshape(NCELL, 1, 33)
    w3r = weight3.reshape(NCELL, 32, 32)
    b3r = bias3.reshape(NCELL, 1, 32)
    w4r = weight4.reshape(NCELL, 59, 32)
    b4r = bias4.reshape(NCELL, 1, 32)
    w5r = weight5.reshape(NCELL, 32, 3)
    b5r = bias5.reshape(NCELL, 1, 3)

    def pspec(k, n):
        return pl.BlockSpec((CH, k, n), lambda g: (g, 0, 0))

    big = pl.pallas_call(
        _pack_body,
        grid=(NCELL // CH,),
        in_specs=[pspec(63, 32), pspec(1, 32), pspec(32, 33), pspec(1, 33),
                  pspec(32, 32), pspec(1, 32), pspec(59, 32), pspec(1, 32),
                  pspec(32, 3), pspec(1, 3)],
        out_specs=pl.BlockSpec((CH, 80, 128), lambda g: (g, 0, 0)),
        out_shape=jax.ShapeDtypeStruct((NCELL, 80, 128), jnp.float32),
    )(w1r, b1r, w2r, b2r, w3r, b3r, w4r, b4r, w5r, b5r)

    half = SCALE / 2
    mask = (jnp.abs(x[:, 0]) < half) & (jnp.abs(x[:, 1]) < half) & (jnp.abs(x[:, 2]) < half)
    i = jnp.clip((x / (SCALE / N) + N / 2).astype(jnp.int32), 0, N - 1)
    c = (i[:, 0] * N + i[:, 1]) * N + i[:, 2]
    c = jnp.where(mask, c, NCELL).astype(jnp.int32)

    row = jnp.arange(B, dtype=jnp.int32)
    cs, order = lax.sort((c, row), num_keys=1)
    xs = x[order]
    ds = d[order]

    prev = jnp.concatenate([jnp.full((1,), -1, jnp.int32), cs[:-1]])
    is_new = ((row % T) == 0) | (cs != prev)
    bpos = jnp.where(is_new, row, B)
    rstart = lax.sort(bpos)[:W_ITEMS]
    rnext = jnp.concatenate([rstart[1:], jnp.full((1,), B, jnp.int32)])
    valid_it = rstart < B
    s_tile = jnp.minimum(rstart // T, NTILES - 1)
    s_s = rstart % T
    s_e = jnp.where(valid_it, ((rnext - 1) % T) + 1, 0)
    s_cid = cs[jnp.minimum(rstart, B - 1)]
    s_first = ((s_s == 0) & valid_it).astype(jnp.int32)

    enc = pl.pallas_call(
        _enc_body,
        grid=(B // TE,),
        in_specs=[pl.BlockSpec((TE, 3), lambda t: (t, 0)),
                  pl.BlockSpec((TE, 3), lambda t: (t, 0))],
        out_specs=[pl.BlockSpec((TE, DX), lambda t: (t, 0)),
                   pl.BlockSpec((TE, DD), lambda t: (t, 0))],
        out_shape=[jax.ShapeDtypeStruct((B, DX), jnp.float32),
                   jax.ShapeDtypeStruct((B, DD), jnp.float32)],
    )
    ex, ed = enc(xs, ds)

    grid_spec = pltpu.PrefetchScalarGridSpec(
        num_scalar_prefetch=5,
        grid=(W_ITEMS,),
        in_specs=[
            pl.BlockSpec((T, DX), lambda w, st, sc, ss, se, sf: (st[w], 0)),
            pl.BlockSpec((T, DD), lambda w, st, sc, ss, se, sf: (st[w], 0)),
            pl.BlockSpec((1, 80, 128),
                         lambda w, st, sc, ss, se, sf:
                         (jnp.minimum(sc[w], NCELL - 1), 0, 0)),
        ],
        out_specs=pl.BlockSpec((T, 8), lambda w, st, sc, ss, se, sf: (st[w], 0)),
    )
    out_s = pl.pallas_call(
        _mlp_body,
        grid_spec=grid_spec,
        out_shape=jax.ShapeDtypeStruct((B, 8), jnp.float32),
    )(s_tile, s_cid, s_s, s_e, s_first, ex, ed, big)

    out = jnp.zeros((B, 8), jnp.float32).at[order].set(out_s)
    return out[:, 0:3], out[:, 3:4]
